# Initial kernel scaffold; baseline (speedup 1.0000x reference)
import jax
import jax.numpy as jnp
from jax.experimental import pallas as pl


def kernel(x_categorical, x_numerical, emb_tables, bn_num_g, bn_num_b, W1, b1, g1, be1, W2, b2, g2, be2, W3, b3):
    raise NotImplementedError("write your pallas kernel here")


# speedup vs baseline: 2.1927x; 2.1927x over previous
import functools

import jax
import jax.numpy as jnp
from jax import lax
from jax.experimental import pallas as pl
from jax.experimental.pallas import tpu as pltpu
from jax.experimental.pallas import tpu_sc as plsc

B = 4096
F = 26
V = 100000
D = 32
NUM = 13
H1 = 512
H2 = 256
OUT = 100
EPS = 1e-5

NC = 2
NS = 16
NW = NC * NS
ROWS = B * F
RPW = ROWS // NW
CHUNK = 128
NCHUNK = RPW // CHUNK


def _gather_call(table, idx):
    mesh = plsc.VectorSubcoreMesh(
        core_axis_name="c", subcore_axis_name="s", num_cores=NC, num_subcores=NS
    )

    @functools.partial(
        pl.kernel,
        mesh=mesh,
        compiler_params=pltpu.CompilerParams(use_tc_tiling_on_sc=False),
        out_type=jax.ShapeDtypeStruct((NW, NCHUNK, CHUNK, D), jnp.float32),
        scratch_types=[
            pltpu.VMEM((NCHUNK, CHUNK), jnp.int32),
            pltpu.VMEM((NCHUNK, CHUNK, D), jnp.float32),
            pltpu.SemaphoreType.DMA,
        ],
    )
    def gather_k(table_hbm, idx_hbm, out_hbm, idx_v, rows_v, sem):
        wid = lax.axis_index("s") * NC + lax.axis_index("c")
        pltpu.sync_copy(idx_hbm.at[wid], idx_v)
        copies = [
            pltpu.async_copy(table_hbm.at[idx_v.at[j]], rows_v.at[j], sem)
            for j in range(NCHUNK)
        ]
        for cp in copies:
            cp.wait()
        pltpu.sync_copy(rows_v, out_hbm.at[wid])

    return gather_k(table, idx)


def _bn(x, g, b):
    m = jnp.mean(x, axis=0, keepdims=True)
    v = jnp.mean((x - m) * (x - m), axis=0, keepdims=True)
    return g * (x - m) / jnp.sqrt(v + EPS) + b


def _mlp_body(emb_ref, xn_ref, bng_ref, bnb_ref, w1e_ref, w1n_ref, b1_ref,
              g1_ref, be1_ref, w2_ref, b2_ref, g2_ref, be2_ref, w3_ref,
              b3_ref, out_ref):
    dn = (((1,), (1,)), ((), ()))
    xnb = _bn(xn_ref[:], bng_ref[:], bnb_ref[:])
    h = lax.dot_general(emb_ref[:], w1e_ref[:], dn,
                        preferred_element_type=jnp.float32)
    h = h + lax.dot_general(xnb, w1n_ref[:], dn,
                            preferred_element_type=jnp.float32)
    h = jnp.maximum(h + b1_ref[:], 0.0)
    h = _bn(h, g1_ref[:], be1_ref[:])
    h = lax.dot_general(h, w2_ref[:], dn, preferred_element_type=jnp.float32)
    h = jnp.maximum(h + b2_ref[:], 0.0)
    h = _bn(h, g2_ref[:], be2_ref[:])
    out_ref[:] = (
        lax.dot_general(h, w3_ref[:], dn, preferred_element_type=jnp.float32)
        + b3_ref[:]
    )


def kernel(x_categorical, x_numerical, emb_tables, bn_num_g, bn_num_b,
           W1, b1, g1, be1, W2, b2, g2, be2, W3, b3):
    offs = (jnp.arange(F, dtype=jnp.int32) * V)[None, :]
    flat_idx = (x_categorical.astype(jnp.int32) + offs).reshape(NW, NCHUNK, CHUNK)
    table = emb_tables.reshape(F * V, D)
    emb = _gather_call(table, flat_idx).reshape(B, F * D)

    out = pl.pallas_call(
        _mlp_body,
        out_shape=jax.ShapeDtypeStruct((B, OUT), jnp.float32),
    )(
        emb,
        x_numerical,
        bn_num_g[None, :],
        bn_num_b[None, :],
        W1[:, : F * D],
        W1[:, F * D:],
        b1[None, :],
        g1[None, :],
        be1[None, :],
        W2,
        b2[None, :],
        g2[None, :],
        be2[None, :],
        W3,
        b3[None, :],
    )
    return out
